# baseline (speedup 1.0000x reference)
import functools

import jax
import jax.numpy as jnp
from jax import lax
from jax.experimental import pallas as pl
from jax.experimental.pallas import tpu as pltpu
from jax.experimental.pallas import tpu_sc as plsc

N = 10000
E = 320000
D = 128
K = 10
ALPHA = 0.1

NPAD = 10240
W = 64
NC, NS = 2, 16
NWORK = NC * NS
NB = NPAD // NWORK
JUNK = NB
EPAD = E + W + 8
MAXWIN = (E + 8 + W - 1) // W + 1


def _mesh():
    return plsc.VectorSubcoreMesh(core_axis_name="c", subcore_axis_name="s")


def _sread(ref, i):
    return ref[pl.ds(i, 16)][0]

def _sc_step(g, srcp, dstl, bounds, nrm, f0):
    out_t = (jax.ShapeDtypeStruct((NPAD, D), jnp.float32),
             jax.ShapeDtypeStruct((NPAD, D), jnp.float32))

    @functools.partial(
        pl.kernel, mesh=_mesh(), out_type=out_t,
        scratch_types=[
            pltpu.VMEM((W,), jnp.int32),
            pltpu.VMEM((W,), jnp.int32),
            pltpu.VMEM((W + 16,), jnp.int32),
            pltpu.VMEM((W + 16,), jnp.int32),
            pltpu.VMEM((W, D), jnp.float32),
            pltpu.VMEM((W, D), jnp.float32),
            pltpu.VMEM((NB, D), jnp.float32),
            pltpu.VMEM((NWORK + 16,), jnp.int32),
            pltpu.VMEM((NB, 16), jnp.float32),
            pltpu.SemaphoreType.DMA,
            pltpu.SemaphoreType.DMA,
            pltpu.SemaphoreType.DMA,
            pltpu.SemaphoreType.DMA,
            pltpu.SemaphoreType.DMA,
            pltpu.SemaphoreType.DMA,
        ],
    )
    def k(g_hbm, src_hbm, dstl_hbm, bnd_hbm, nrm_hbm,
          f0_hbm, gout_hbm, hout_hbm,
          is0, is1, dl0, dl1, r0, r1, acc, bnd, nv,
          ss0, ss1, sd0, sd1, sg0, sg1):
        idxs = (is0, is1)
        dlss = (dl0, dl1)
        rows = (r0, r1)
        ssem = (ss0, ss1)
        dsem = (sd0, sd1)
        gsem = (sg0, sg1)

        c = lax.axis_index("c")
        s = lax.axis_index("s")
        w = s * NC + c

        pltpu.sync_copy(bnd_hbm, bnd)
        start = _sread(bnd, w)
        end = _sread(bnd, w + 1)
        base = (start // 8) * 8
        nwin = (end - base + (W - 1)) // W

        def start_idx(t, b):
            p = base + t * W
            pltpu.async_copy(src_hbm.at[pl.ds(p, W)], idxs[b], ssem[b])
            pltpu.async_copy(dstl_hbm.at[pl.ds(p, W)], dlss[b].at[pl.ds(0, W)],
                             dsem[b])

        def wait_idx(b):
            pltpu.make_async_copy(src_hbm.at[pl.ds(0, W)], idxs[b],
                                  ssem[b]).wait()
            pltpu.make_async_copy(dstl_hbm.at[pl.ds(0, W)],
                                  dlss[b].at[pl.ds(0, W)], dsem[b]).wait()

        def start_gather(b):
            pltpu.async_copy(g_hbm.at[idxs[b]], rows[b], gsem[b])

        def wait_gather(b):
            pltpu.make_async_copy(g_hbm.at[idxs[b]], rows[b], gsem[b]).wait()

        def accum_edge(b, e):
            dl = _sread(dlss[b], e)
            for q in range(D // 16):
                sl = pl.ds(q * 16, 16)
                plsc.addupdate(acc.at[dl, sl], rows[b][e, sl])

        def accumulate(b, t):
            p = base + t * W
            interior = (p >= start) & (p + W <= end)

            @pl.when(interior)
            def _():
                @pl.loop(0, W)
                def _(e):
                    accum_edge(b, e)

            @pl.when(jnp.logical_not(interior))
            def _():
                @pl.loop(0, W)
                def _(e):
                    pos = p + e

                    @pl.when((pos >= start) & (pos < end))
                    def _():
                        accum_edge(b, e)

        @pl.loop(0, NB)
        def _(r):
            @pl.loop(0, D // 16)
            def _(q):
                acc[r, pl.ds(q * 16, 16)] = jnp.zeros((16,), jnp.float32)

        @pl.when(nwin > 0)
        def _():
            start_idx(0, 0)
            wait_idx(0)
            start_gather(0)

        @pl.when(nwin > 1)
        def _():
            start_idx(1, 1)

        @pl.loop(0, (MAXWIN + 1) // 2)
        def _(gi):
            for b in (0, 1):
                t = gi * 2 + b

                @pl.when(t < nwin)
                def _(t=t, b=b):
                    wait_gather(b)

                    @pl.when(t + 1 < nwin)
                    def _():
                        wait_idx(1 - b)
                        start_gather(1 - b)

                    accumulate(b, t)

                    @pl.when(t + 2 < nwin)
                    def _():
                        start_idx(t + 2, b)

        pltpu.sync_copy(nrm_hbm.at[pl.ds(w * NB, NB)], nv)

        @pl.loop(0, NB // 64)
        def _(ch):
            r0c = ch * 64
            pltpu.sync_copy(f0_hbm.at[pl.ds(w * NB + r0c, 64)], r0)

            @pl.loop(0, 64)
            def _(r):
                row = r0c + r
                nvr = nv[row, :]
                snr = nvr[0]
                dnr = nvr[8]
                for q in range(D // 16):
                    sl = pl.ds(q * 16, 16)
                    hq = ((1.0 - ALPHA) * dnr) * acc[row, sl] + ALPHA * r0[r, sl]
                    r0[r, sl] = hq
                    acc[row, sl] = snr * hq

            pltpu.sync_copy(r0, hout_hbm.at[pl.ds(w * NB + r0c, 64)])

        pltpu.sync_copy(acc, gout_hbm.at[pl.ds(w * NB, NB)])

    return k(g, srcp, dstl, bounds, nrm, f0)


def _sc_count(nodel, bounds):
    out_t = jax.ShapeDtypeStruct((NPAD, 16), jnp.float32)

    @functools.partial(
        pl.kernel, mesh=_mesh(), out_type=out_t,
        scratch_types=[
            pltpu.VMEM((NB, 16), jnp.float32),
            pltpu.VMEM((W + 16,), jnp.int32),
            pltpu.VMEM((NWORK + 16,), jnp.int32),
        ],
    )
    def k(nodel_hbm, bnd_hbm, deg_hbm, acc, dls, bnd):
        c = lax.axis_index("c")
        s = lax.axis_index("s")
        w = s * NC + c

        pltpu.sync_copy(bnd_hbm, bnd)
        start = _sread(bnd, w)
        end = _sread(bnd, w + 1)
        base = (start // 8) * 8
        nwin = (end - base + (W - 1)) // W

        @pl.loop(0, NB)
        def _(r):
            acc[r, :] = jnp.zeros((16,), jnp.float32)

        @pl.loop(0, MAXWIN)
        def _(t):
            @pl.when(t < nwin)
            def _():
                p = base + t * W
                pltpu.sync_copy(nodel_hbm.at[pl.ds(p, W)], dls.at[pl.ds(0, W)])

                @pl.loop(0, W)
                def _(e):
                    pos = p + e

                    @pl.when((pos >= start) & (pos < end))
                    def _():
                        dl = _sread(dls, e)
                        plsc.addupdate(acc.at[dl], jnp.ones((16,), jnp.float32))

        pltpu.sync_copy(acc, deg_hbm.at[pl.ds(w * NB, NB)])

    return k(nodel, bounds)


_BN = 256


def _tc_norms(odeg, ideg, feat0p):
    def body(od_ref, id_ref, f0_ref, nrm_ref, g0_ref):
        sn = 1.0 / jnp.sqrt(jnp.clip(od_ref[...], 1.0, None))
        dn = 1.0 / jnp.sqrt(jnp.clip(id_ref[...], 1.0, None))
        nrm_ref[...] = jnp.concatenate([sn[:, :8], dn[:, 8:]], axis=1)
        g0_ref[...] = f0_ref[...] * sn[:, 0:1]

    return pl.pallas_call(
        body,
        grid=(NPAD // _BN,),
        in_specs=[
            pl.BlockSpec((_BN, 16), lambda i: (i, 0)),
            pl.BlockSpec((_BN, 16), lambda i: (i, 0)),
            pl.BlockSpec((_BN, D), lambda i: (i, 0)),
        ],
        out_specs=[
            pl.BlockSpec((_BN, 16), lambda i: (i, 0)),
            pl.BlockSpec((_BN, D), lambda i: (i, 0)),
        ],
        out_shape=[
            jax.ShapeDtypeStruct((NPAD, 16), jnp.float32),
            jax.ShapeDtypeStruct((NPAD, D), jnp.float32),
        ],
    )(odeg, ideg, feat0p)


def _tc_combine(p, feat0p, sn, dn):
    def body(p_ref, f0_ref, sn_ref, dn_ref, g_ref, h_ref):
        h = (1.0 - ALPHA) * (p_ref[...] * dn_ref[:, 0:1]) + ALPHA * f0_ref[...]
        h_ref[...] = h
        g_ref[...] = h * sn_ref[:, 0:1]

    return pl.pallas_call(
        body,
        grid=(NPAD // _BN,),
        in_specs=[
            pl.BlockSpec((_BN, D), lambda i: (i, 0)),
            pl.BlockSpec((_BN, D), lambda i: (i, 0)),
            pl.BlockSpec((_BN, 16), lambda i: (i, 0)),
            pl.BlockSpec((_BN, 16), lambda i: (i, 0)),
        ],
        out_specs=[
            pl.BlockSpec((_BN, D), lambda i: (i, 0)),
            pl.BlockSpec((_BN, D), lambda i: (i, 0)),
        ],
        out_shape=[
            jax.ShapeDtypeStruct((NPAD, D), jnp.float32),
            jax.ShapeDtypeStruct((NPAD, D), jnp.float32),
        ],
    )(p, feat0p, sn, dn)


def _layout(key_nodes, other=None):
    order = jnp.argsort(key_nodes)
    keys = key_nodes[order]
    keyl = (keys - (keys // NB) * NB).astype(jnp.int32)
    npad = EPAD - E
    keyl_p = jnp.concatenate([keyl, jnp.zeros((npad,), jnp.int32)])
    bounds = jnp.searchsorted(keys, jnp.arange(NWORK + 1, dtype=jnp.int32) * NB)
    bounds = jnp.pad(bounds.astype(jnp.int32), (0, 15), mode="edge")
    if other is None:
        return keyl_p, bounds
    oth_p = jnp.concatenate([other[order], jnp.full((npad,), N, jnp.int32)])
    return oth_p, keyl_p, bounds


def kernel(feat, edge_index):
    src = edge_index[0].astype(jnp.int32)
    dst = edge_index[1].astype(jnp.int32)

    srcp, dstl, bounds_d = _layout(dst, src)
    srcl, bounds_s = _layout(src)
    feat0p = jnp.pad(feat, ((0, NPAD - N), (0, 0)))

    odeg = _sc_count(srcl, bounds_s)
    ideg = _sc_count(dstl, bounds_d)
    nrm, g = _tc_norms(odeg, ideg, feat0p)

    h = feat0p
    for _ in range(K):
        g, h = _sc_step(g, srcp, dstl, bounds_d, nrm, feat0p)
    return h[:N]


# speedup vs baseline: 1.0938x; 1.0938x over previous
import functools

import jax
import jax.numpy as jnp
from jax import lax
from jax.experimental import pallas as pl
from jax.experimental.pallas import tpu as pltpu
from jax.experimental.pallas import tpu_sc as plsc

N = 10000
E = 320000
D = 128
K = 10
ALPHA = 0.1

NPAD = 10240
W = 64
NC, NS = 2, 16
NWORK = NC * NS
NB = NPAD // NWORK
JUNK = NB
EPAD = E + W + 8
MAXWIN = (E + 8 + W - 1) // W + 1


def _mesh():
    return plsc.VectorSubcoreMesh(core_axis_name="c", subcore_axis_name="s")


def _sread(ref, i):
    return ref[pl.ds(i, 16)][0]

def _sc_step(g, srcp, dstl, bounds, nrm, f0):
    out_t = (jax.ShapeDtypeStruct((NPAD, D), jnp.float32),
             jax.ShapeDtypeStruct((NPAD, D), jnp.float32))

    @functools.partial(
        pl.kernel, mesh=_mesh(), out_type=out_t,
        scratch_types=[
            pltpu.VMEM((W,), jnp.int32),
            pltpu.VMEM((W,), jnp.int32),
            pltpu.VMEM((W + 16,), jnp.int32),
            pltpu.VMEM((W + 16,), jnp.int32),
            pltpu.VMEM((W, D), jnp.float32),
            pltpu.VMEM((W, D), jnp.float32),
            pltpu.VMEM((NB, D), jnp.float32),
            pltpu.VMEM((NWORK + 16,), jnp.int32),
            pltpu.VMEM((NB, 16), jnp.float32),
            pltpu.SemaphoreType.DMA,
            pltpu.SemaphoreType.DMA,
            pltpu.SemaphoreType.DMA,
            pltpu.SemaphoreType.DMA,
            pltpu.SemaphoreType.DMA,
            pltpu.SemaphoreType.DMA,
        ],
    )
    def k(g_hbm, src_hbm, dstl_hbm, bnd_hbm, nrm_hbm,
          f0_hbm, gout_hbm, hout_hbm,
          is0, is1, dl0, dl1, r0, r1, acc, bnd, nv,
          ss0, ss1, sd0, sd1, sg0, sg1):
        idxs = (is0, is1)
        dlss = (dl0, dl1)
        rows = (r0, r1)
        ssem = (ss0, ss1)
        dsem = (sd0, sd1)
        gsem = (sg0, sg1)

        c = lax.axis_index("c")
        s = lax.axis_index("s")
        w = s * NC + c

        pltpu.sync_copy(bnd_hbm, bnd)
        start = _sread(bnd, w)
        end = _sread(bnd, w + 1)
        base = (start // 8) * 8
        nwin = (end - base + (W - 1)) // W

        def start_idx(t, b):
            p = base + t * W
            pltpu.async_copy(src_hbm.at[pl.ds(p, W)], idxs[b], ssem[b])
            pltpu.async_copy(dstl_hbm.at[pl.ds(p, W)], dlss[b].at[pl.ds(0, W)],
                             dsem[b])

        def wait_idx(b):
            pltpu.make_async_copy(src_hbm.at[pl.ds(0, W)], idxs[b],
                                  ssem[b]).wait()
            pltpu.make_async_copy(dstl_hbm.at[pl.ds(0, W)],
                                  dlss[b].at[pl.ds(0, W)], dsem[b]).wait()

        def start_gather(b):
            pltpu.async_copy(g_hbm.at[idxs[b]], rows[b], gsem[b])

        def wait_gather(b):
            pltpu.make_async_copy(g_hbm.at[idxs[b]], rows[b], gsem[b]).wait()

        def accum_edge(b, e):
            dl = _sread(dlss[b], e)
            for q in range(D // 16):
                sl = pl.ds(q * 16, 16)
                plsc.addupdate(acc.at[dl, sl], rows[b][e, sl])

        def flush(dl, vecs):
            for q in range(D // 16):
                plsc.addupdate(acc.at[dl, pl.ds(q * 16, 16)], vecs[q])

        def accum_run(b):
            prev = _sread(dlss[b], 0)
            car = [rows[b][0, pl.ds(q * 16, 16)] for q in range(D // 16)]
            for e in range(1, W):
                dl = _sread(dlss[b], e)
                changed = dl != prev
                keep = jnp.where(changed, 0.0, 1.0)
                keep_v = jnp.full((16,), keep, jnp.float32)

                @pl.when(changed)
                def _(prev=prev, car=tuple(car)):
                    flush(prev, car)

                for q in range(D // 16):
                    rq = rows[b][e, pl.ds(q * 16, 16)]
                    car[q] = rq + car[q] * keep_v
                prev = dl
            flush(prev, tuple(car))

        def accumulate(b, t):
            p = base + t * W
            interior = (p >= start) & (p + W <= end)

            @pl.when(interior)
            def _():
                accum_run(b)

            @pl.when(jnp.logical_not(interior))
            def _():
                @pl.loop(0, W)
                def _(e):
                    pos = p + e

                    @pl.when((pos >= start) & (pos < end))
                    def _():
                        accum_edge(b, e)

        @pl.loop(0, NB)
        def _(r):
            @pl.loop(0, D // 16)
            def _(q):
                acc[r, pl.ds(q * 16, 16)] = jnp.zeros((16,), jnp.float32)

        @pl.when(nwin > 0)
        def _():
            start_idx(0, 0)
            wait_idx(0)
            start_gather(0)

        @pl.when(nwin > 1)
        def _():
            start_idx(1, 1)

        @pl.loop(0, (MAXWIN + 1) // 2)
        def _(gi):
            for b in (0, 1):
                t = gi * 2 + b

                @pl.when(t < nwin)
                def _(t=t, b=b):
                    wait_gather(b)

                    @pl.when(t + 1 < nwin)
                    def _():
                        wait_idx(1 - b)
                        start_gather(1 - b)

                    accumulate(b, t)

                    @pl.when(t + 2 < nwin)
                    def _():
                        start_idx(t + 2, b)

        pltpu.sync_copy(nrm_hbm.at[pl.ds(w * NB, NB)], nv)

        @pl.loop(0, NB // 64)
        def _(ch):
            r0c = ch * 64
            pltpu.sync_copy(f0_hbm.at[pl.ds(w * NB + r0c, 64)], r0)

            @pl.loop(0, 64)
            def _(r):
                row = r0c + r
                nvr = nv[row, :]
                snr = nvr[0]
                dnr = nvr[8]
                for q in range(D // 16):
                    sl = pl.ds(q * 16, 16)
                    hq = ((1.0 - ALPHA) * dnr) * acc[row, sl] + ALPHA * r0[r, sl]
                    r0[r, sl] = hq
                    acc[row, sl] = snr * hq

            pltpu.sync_copy(r0, hout_hbm.at[pl.ds(w * NB + r0c, 64)])

        pltpu.sync_copy(acc, gout_hbm.at[pl.ds(w * NB, NB)])

    return k(g, srcp, dstl, bounds, nrm, f0)


def _sc_count(nodel, bounds):
    out_t = jax.ShapeDtypeStruct((NPAD, 16), jnp.float32)

    @functools.partial(
        pl.kernel, mesh=_mesh(), out_type=out_t,
        scratch_types=[
            pltpu.VMEM((NB, 16), jnp.float32),
            pltpu.VMEM((W + 16,), jnp.int32),
            pltpu.VMEM((NWORK + 16,), jnp.int32),
        ],
    )
    def k(nodel_hbm, bnd_hbm, deg_hbm, acc, dls, bnd):
        c = lax.axis_index("c")
        s = lax.axis_index("s")
        w = s * NC + c

        pltpu.sync_copy(bnd_hbm, bnd)
        start = _sread(bnd, w)
        end = _sread(bnd, w + 1)
        base = (start // 8) * 8
        nwin = (end - base + (W - 1)) // W

        @pl.loop(0, NB)
        def _(r):
            acc[r, :] = jnp.zeros((16,), jnp.float32)

        @pl.loop(0, MAXWIN)
        def _(t):
            @pl.when(t < nwin)
            def _():
                p = base + t * W
                pltpu.sync_copy(nodel_hbm.at[pl.ds(p, W)], dls.at[pl.ds(0, W)])

                @pl.loop(0, W)
                def _(e):
                    pos = p + e

                    @pl.when((pos >= start) & (pos < end))
                    def _():
                        dl = _sread(dls, e)
                        plsc.addupdate(acc.at[dl], jnp.ones((16,), jnp.float32))

        pltpu.sync_copy(acc, deg_hbm.at[pl.ds(w * NB, NB)])

    return k(nodel, bounds)


_BN = 256


def _tc_norms(odeg, ideg, feat0p):
    def body(od_ref, id_ref, f0_ref, nrm_ref, g0_ref):
        sn = 1.0 / jnp.sqrt(jnp.clip(od_ref[...], 1.0, None))
        dn = 1.0 / jnp.sqrt(jnp.clip(id_ref[...], 1.0, None))
        nrm_ref[...] = jnp.concatenate([sn[:, :8], dn[:, 8:]], axis=1)
        g0_ref[...] = f0_ref[...] * sn[:, 0:1]

    return pl.pallas_call(
        body,
        grid=(NPAD // _BN,),
        in_specs=[
            pl.BlockSpec((_BN, 16), lambda i: (i, 0)),
            pl.BlockSpec((_BN, 16), lambda i: (i, 0)),
            pl.BlockSpec((_BN, D), lambda i: (i, 0)),
        ],
        out_specs=[
            pl.BlockSpec((_BN, 16), lambda i: (i, 0)),
            pl.BlockSpec((_BN, D), lambda i: (i, 0)),
        ],
        out_shape=[
            jax.ShapeDtypeStruct((NPAD, 16), jnp.float32),
            jax.ShapeDtypeStruct((NPAD, D), jnp.float32),
        ],
    )(odeg, ideg, feat0p)


def _tc_combine(p, feat0p, sn, dn):
    def body(p_ref, f0_ref, sn_ref, dn_ref, g_ref, h_ref):
        h = (1.0 - ALPHA) * (p_ref[...] * dn_ref[:, 0:1]) + ALPHA * f0_ref[...]
        h_ref[...] = h
        g_ref[...] = h * sn_ref[:, 0:1]

    return pl.pallas_call(
        body,
        grid=(NPAD // _BN,),
        in_specs=[
            pl.BlockSpec((_BN, D), lambda i: (i, 0)),
            pl.BlockSpec((_BN, D), lambda i: (i, 0)),
            pl.BlockSpec((_BN, 16), lambda i: (i, 0)),
            pl.BlockSpec((_BN, 16), lambda i: (i, 0)),
        ],
        out_specs=[
            pl.BlockSpec((_BN, D), lambda i: (i, 0)),
            pl.BlockSpec((_BN, D), lambda i: (i, 0)),
        ],
        out_shape=[
            jax.ShapeDtypeStruct((NPAD, D), jnp.float32),
            jax.ShapeDtypeStruct((NPAD, D), jnp.float32),
        ],
    )(p, feat0p, sn, dn)


def _layout(key_nodes, other=None):
    order = jnp.argsort(key_nodes)
    keys = key_nodes[order]
    keyl = (keys - (keys // NB) * NB).astype(jnp.int32)
    npad = EPAD - E
    keyl_p = jnp.concatenate([keyl, jnp.zeros((npad,), jnp.int32)])
    bounds = jnp.searchsorted(keys, jnp.arange(NWORK + 1, dtype=jnp.int32) * NB)
    bounds = jnp.pad(bounds.astype(jnp.int32), (0, 15), mode="edge")
    if other is None:
        return keyl_p, bounds
    oth_p = jnp.concatenate([other[order], jnp.full((npad,), N, jnp.int32)])
    return oth_p, keyl_p, bounds


def kernel(feat, edge_index):
    src = edge_index[0].astype(jnp.int32)
    dst = edge_index[1].astype(jnp.int32)

    srcp, dstl, bounds_d = _layout(dst, src)
    srcl, bounds_s = _layout(src)
    feat0p = jnp.pad(feat, ((0, NPAD - N), (0, 0)))

    odeg = _sc_count(srcl, bounds_s)
    ideg = _sc_count(dstl, bounds_d)
    nrm, g = _tc_norms(odeg, ideg, feat0p)

    h = feat0p
    for _ in range(K):
        g, h = _sc_step(g, srcp, dstl, bounds_d, nrm, feat0p)
    return h[:N]


# speedup vs baseline: 1.1405x; 1.0427x over previous
import functools

import jax
import jax.numpy as jnp
from jax import lax
from jax.experimental import pallas as pl
from jax.experimental.pallas import tpu as pltpu
from jax.experimental.pallas import tpu_sc as plsc

N = 10000
E = 320000
D = 128
K = 10
ALPHA = 0.1

NPAD = 10240
W = 96
NC, NS = 2, 16
NWORK = NC * NS
NB = NPAD // NWORK
JUNK = NB
EPAD = E + W + 8
MAXWIN = (E + 8 + W - 1) // W + 1


def _mesh():
    return plsc.VectorSubcoreMesh(core_axis_name="c", subcore_axis_name="s")


def _sread(ref, i):
    return ref[pl.ds(i, 16)][0]

def _sc_step(g, srcp, dstl, bounds, nrm, f0):
    out_t = (jax.ShapeDtypeStruct((NPAD, D), jnp.float32),
             jax.ShapeDtypeStruct((NPAD, D), jnp.float32))

    @functools.partial(
        pl.kernel, mesh=_mesh(), out_type=out_t,
        scratch_types=[
            pltpu.VMEM((W,), jnp.int32),
            pltpu.VMEM((W,), jnp.int32),
            pltpu.VMEM((W + 16,), jnp.int32),
            pltpu.VMEM((W + 16,), jnp.int32),
            pltpu.VMEM((W, D), jnp.float32),
            pltpu.VMEM((W, D), jnp.float32),
            pltpu.VMEM((NB, D), jnp.float32),
            pltpu.VMEM((NWORK + 16,), jnp.int32),
            pltpu.VMEM((NB, 16), jnp.float32),
            pltpu.SemaphoreType.DMA,
            pltpu.SemaphoreType.DMA,
            pltpu.SemaphoreType.DMA,
            pltpu.SemaphoreType.DMA,
            pltpu.SemaphoreType.DMA,
            pltpu.SemaphoreType.DMA,
        ],
    )
    def k(g_hbm, src_hbm, dstl_hbm, bnd_hbm, nrm_hbm,
          f0_hbm, gout_hbm, hout_hbm,
          is0, is1, dl0, dl1, r0, r1, acc, bnd, nv,
          ss0, ss1, sd0, sd1, sg0, sg1):
        idxs = (is0, is1)
        dlss = (dl0, dl1)
        rows = (r0, r1)
        ssem = (ss0, ss1)
        dsem = (sd0, sd1)
        gsem = (sg0, sg1)

        c = lax.axis_index("c")
        s = lax.axis_index("s")
        w = s * NC + c

        pltpu.sync_copy(bnd_hbm, bnd)
        start = _sread(bnd, w)
        end = _sread(bnd, w + 1)
        base = (start // 8) * 8
        nwin = (end - base + (W - 1)) // W

        def start_idx(t, b):
            p = base + t * W
            pltpu.async_copy(src_hbm.at[pl.ds(p, W)], idxs[b], ssem[b])
            pltpu.async_copy(dstl_hbm.at[pl.ds(p, W)], dlss[b].at[pl.ds(0, W)],
                             dsem[b])

        def wait_idx(b):
            pltpu.make_async_copy(src_hbm.at[pl.ds(0, W)], idxs[b],
                                  ssem[b]).wait()
            pltpu.make_async_copy(dstl_hbm.at[pl.ds(0, W)],
                                  dlss[b].at[pl.ds(0, W)], dsem[b]).wait()

        def start_gather(b):
            pltpu.async_copy(g_hbm.at[idxs[b]], rows[b], gsem[b])

        def wait_gather(b):
            pltpu.make_async_copy(g_hbm.at[idxs[b]], rows[b], gsem[b]).wait()

        def accum_edge(b, e):
            dl = _sread(dlss[b], e)
            for q in range(D // 16):
                sl = pl.ds(q * 16, 16)
                plsc.addupdate(acc.at[dl, sl], rows[b][e, sl])

        def flush(dl, vecs):
            for q in range(D // 16):
                plsc.addupdate(acc.at[dl, pl.ds(q * 16, 16)], vecs[q])

        def accum_run(b):
            prev = _sread(dlss[b], 0)
            car = [rows[b][0, pl.ds(q * 16, 16)] for q in range(D // 16)]
            for e in range(1, W):
                dl = _sread(dlss[b], e)
                changed = dl != prev
                keep = jnp.where(changed, 0.0, 1.0)
                keep_v = jnp.full((16,), keep, jnp.float32)

                @pl.when(changed)
                def _(prev=prev, car=tuple(car)):
                    flush(prev, car)

                for q in range(D // 16):
                    rq = rows[b][e, pl.ds(q * 16, 16)]
                    car[q] = rq + car[q] * keep_v
                prev = dl
            flush(prev, tuple(car))

        def accumulate(b, t):
            p = base + t * W
            interior = (p >= start) & (p + W <= end)

            @pl.when(interior)
            def _():
                accum_run(b)

            @pl.when(jnp.logical_not(interior))
            def _():
                @pl.loop(0, W)
                def _(e):
                    pos = p + e

                    @pl.when((pos >= start) & (pos < end))
                    def _():
                        accum_edge(b, e)

        @pl.loop(0, NB)
        def _(r):
            @pl.loop(0, D // 16)
            def _(q):
                acc[r, pl.ds(q * 16, 16)] = jnp.zeros((16,), jnp.float32)

        @pl.when(nwin > 0)
        def _():
            start_idx(0, 0)
            wait_idx(0)
            start_gather(0)

        @pl.when(nwin > 1)
        def _():
            start_idx(1, 1)

        @pl.loop(0, (MAXWIN + 1) // 2)
        def _(gi):
            for b in (0, 1):
                t = gi * 2 + b

                @pl.when(t < nwin)
                def _(t=t, b=b):
                    wait_gather(b)

                    @pl.when(t + 1 < nwin)
                    def _():
                        wait_idx(1 - b)
                        start_gather(1 - b)

                    accumulate(b, t)

                    @pl.when(t + 2 < nwin)
                    def _():
                        start_idx(t + 2, b)

        pltpu.sync_copy(nrm_hbm.at[pl.ds(w * NB, NB)], nv)

        @pl.loop(0, NB // 64)
        def _(ch):
            r0c = ch * 64
            pltpu.sync_copy(f0_hbm.at[pl.ds(w * NB + r0c, 64)],
                            r0.at[pl.ds(0, 64)])

            @pl.loop(0, 64)
            def _(r):
                row = r0c + r
                nvr = nv[row, :]
                snr = nvr[0]
                dnr = nvr[8]
                for q in range(D // 16):
                    sl = pl.ds(q * 16, 16)
                    hq = ((1.0 - ALPHA) * dnr) * acc[row, sl] + ALPHA * r0[r, sl]
                    r0[r, sl] = hq
                    acc[row, sl] = snr * hq

            pltpu.sync_copy(r0.at[pl.ds(0, 64)],
                            hout_hbm.at[pl.ds(w * NB + r0c, 64)])

        pltpu.sync_copy(acc, gout_hbm.at[pl.ds(w * NB, NB)])

    return k(g, srcp, dstl, bounds, nrm, f0)


def _sc_count(nodel, bounds):
    out_t = jax.ShapeDtypeStruct((NPAD, 16), jnp.float32)

    @functools.partial(
        pl.kernel, mesh=_mesh(), out_type=out_t,
        scratch_types=[
            pltpu.VMEM((NB, 16), jnp.float32),
            pltpu.VMEM((W + 16,), jnp.int32),
            pltpu.VMEM((NWORK + 16,), jnp.int32),
        ],
    )
    def k(nodel_hbm, bnd_hbm, deg_hbm, acc, dls, bnd):
        c = lax.axis_index("c")
        s = lax.axis_index("s")
        w = s * NC + c

        pltpu.sync_copy(bnd_hbm, bnd)
        start = _sread(bnd, w)
        end = _sread(bnd, w + 1)
        base = (start // 8) * 8
        nwin = (end - base + (W - 1)) // W

        @pl.loop(0, NB)
        def _(r):
            acc[r, :] = jnp.zeros((16,), jnp.float32)

        @pl.loop(0, MAXWIN)
        def _(t):
            @pl.when(t < nwin)
            def _():
                p = base + t * W
                pltpu.sync_copy(nodel_hbm.at[pl.ds(p, W)], dls.at[pl.ds(0, W)])

                @pl.loop(0, W)
                def _(e):
                    pos = p + e

                    @pl.when((pos >= start) & (pos < end))
                    def _():
                        dl = _sread(dls, e)
                        plsc.addupdate(acc.at[dl], jnp.ones((16,), jnp.float32))

        pltpu.sync_copy(acc, deg_hbm.at[pl.ds(w * NB, NB)])

    return k(nodel, bounds)


_BN = 256


def _tc_norms(odeg, ideg, feat0p):
    def body(od_ref, id_ref, f0_ref, nrm_ref, g0_ref):
        sn = 1.0 / jnp.sqrt(jnp.clip(od_ref[...], 1.0, None))
        dn = 1.0 / jnp.sqrt(jnp.clip(id_ref[...], 1.0, None))
        nrm_ref[...] = jnp.concatenate([sn[:, :8], dn[:, 8:]], axis=1)
        g0_ref[...] = f0_ref[...] * sn[:, 0:1]

    return pl.pallas_call(
        body,
        grid=(NPAD // _BN,),
        in_specs=[
            pl.BlockSpec((_BN, 16), lambda i: (i, 0)),
            pl.BlockSpec((_BN, 16), lambda i: (i, 0)),
            pl.BlockSpec((_BN, D), lambda i: (i, 0)),
        ],
        out_specs=[
            pl.BlockSpec((_BN, 16), lambda i: (i, 0)),
            pl.BlockSpec((_BN, D), lambda i: (i, 0)),
        ],
        out_shape=[
            jax.ShapeDtypeStruct((NPAD, 16), jnp.float32),
            jax.ShapeDtypeStruct((NPAD, D), jnp.float32),
        ],
    )(odeg, ideg, feat0p)


def _tc_combine(p, feat0p, sn, dn):
    def body(p_ref, f0_ref, sn_ref, dn_ref, g_ref, h_ref):
        h = (1.0 - ALPHA) * (p_ref[...] * dn_ref[:, 0:1]) + ALPHA * f0_ref[...]
        h_ref[...] = h
        g_ref[...] = h * sn_ref[:, 0:1]

    return pl.pallas_call(
        body,
        grid=(NPAD // _BN,),
        in_specs=[
            pl.BlockSpec((_BN, D), lambda i: (i, 0)),
            pl.BlockSpec((_BN, D), lambda i: (i, 0)),
            pl.BlockSpec((_BN, 16), lambda i: (i, 0)),
            pl.BlockSpec((_BN, 16), lambda i: (i, 0)),
        ],
        out_specs=[
            pl.BlockSpec((_BN, D), lambda i: (i, 0)),
            pl.BlockSpec((_BN, D), lambda i: (i, 0)),
        ],
        out_shape=[
            jax.ShapeDtypeStruct((NPAD, D), jnp.float32),
            jax.ShapeDtypeStruct((NPAD, D), jnp.float32),
        ],
    )(p, feat0p, sn, dn)


def _layout(key_nodes, other=None):
    order = jnp.argsort(key_nodes)
    keys = key_nodes[order]
    keyl = (keys - (keys // NB) * NB).astype(jnp.int32)
    npad = EPAD - E
    keyl_p = jnp.concatenate([keyl, jnp.zeros((npad,), jnp.int32)])
    bounds = jnp.searchsorted(keys, jnp.arange(NWORK + 1, dtype=jnp.int32) * NB)
    bounds = jnp.pad(bounds.astype(jnp.int32), (0, 15), mode="edge")
    if other is None:
        return keyl_p, bounds
    oth_p = jnp.concatenate([other[order], jnp.full((npad,), N, jnp.int32)])
    return oth_p, keyl_p, bounds


def kernel(feat, edge_index):
    src = edge_index[0].astype(jnp.int32)
    dst = edge_index[1].astype(jnp.int32)

    srcp, dstl, bounds_d = _layout(dst, src)
    srcl, bounds_s = _layout(src)
    feat0p = jnp.pad(feat, ((0, NPAD - N), (0, 0)))

    odeg = _sc_count(srcl, bounds_s)
    ideg = _sc_count(dstl, bounds_d)
    nrm, g = _tc_norms(odeg, ideg, feat0p)

    h = feat0p
    for _ in range(K):
        g, h = _sc_step(g, srcp, dstl, bounds_d, nrm, feat0p)
    return h[:N]
